# baseline (speedup 1.0000x reference)
import functools

import jax
import jax.numpy as jnp
from jax import lax
from jax.experimental import pallas as pl
from jax.experimental.pallas import tpu as pltpu
from jax.experimental.pallas import tpu_sc as plsc

N_NODES = 10000
N_EDGES = 320000
D = 128
H = 256
G = 128
NC = 2
NS = 16
CHUNK = 128

EP = 327680
EW1 = EP // (NC * NS)
EW2 = EP // NS
NCH = EP // CHUNK
N_ACC = N_NODES + 112
RPS = N_ACC // NS

R = 2000
NB = N_NODES // R

def _make_sc_agg(feat_split):
    ew = EW2 if feat_split else EW1
    n_chunks = ew // CHUNK
    IDXBLK = 16 if feat_split else 8
    nblk = n_chunks // IDXBLK
    mesh = plsc.VectorSubcoreMesh(core_axis_name="c", subcore_axis_name="s")

    @functools.partial(
        pl.kernel,
        out_type=jax.ShapeDtypeStruct((NC, N_ACC, D), jnp.float32),
        mesh=mesh,
        scratch_types=[
            pltpu.VMEM((2, IDXBLK, CHUNK), jnp.int32),
            pltpu.VMEM((2, IDXBLK, CHUNK), jnp.int32),
            pltpu.VMEM((CHUNK, D), jnp.float32),
            pltpu.VMEM((CHUNK, D), jnp.float32),
            pltpu.VMEM_SHARED((N_ACC, D), jnp.float32),
            pltpu.SemaphoreType.DMA,
            pltpu.SemaphoreType.DMA,
            pltpu.SemaphoreType.DMA,
            pltpu.SemaphoreType.DMA,
        ],
    )
    def sc_agg(table_hbm, src_hbm, dst_hbm, zeros_hbm, out_hbm,
               idx_s, idx_d, rows0, rows1, acc, sem0, sem1, isem0, isem1):
        c = lax.axis_index("c")
        s = lax.axis_index("s")
        rows = (rows0, rows1)
        sems = (sem0, sem1)
        isems = (isem0, isem1)
        if feat_split:
            tbl = table_hbm.at[c]
            row_base = s * n_chunks
        else:
            tbl = table_hbm
            row_base = (c * NS + s) * n_chunks

        def gather(ib, j, b):
            return pltpu.make_async_copy(tbl.at[idx_s.at[ib].at[j]], rows[b],
                                         sems[b])

        def scat(ib, j, b):
            pltpu.sync_copy(rows[b], acc.at[idx_d.at[ib].at[j]], add=True)

        def idx_cp(blk, ib):
            row0 = row_base + blk * IDXBLK
            return (
                pltpu.make_async_copy(src_hbm.at[pl.ds(row0, IDXBLK)],
                                      idx_s.at[ib], isems[ib]),
                pltpu.make_async_copy(dst_hbm.at[pl.ds(row0, IDXBLK)],
                                      idx_d.at[ib], isems[ib]),
            )

        for d in idx_cp(0, 0):
            d.start()
        if nblk > 1:
            for d in idx_cp(1, 1):
                d.start()
        for d in idx_cp(0, 0):
            d.wait()
        gather(0, 0, 0).start()
        gather(0, 1, 1).start()
        pltpu.sync_copy(zeros_hbm.at[pl.ds(s * RPS, RPS)],
                        acc.at[pl.ds(s * RPS, RPS)])
        plsc.subcore_barrier()

        def do_block(blk, ib):
            @pl.loop(0, IDXBLK - 2, step=2)
            def _(k):
                for b in range(2):
                    kk = k + b
                    gather(ib, kk, b).wait()
                    scat(ib, kk, b)
                    gather(ib, kk + 2, b).start()

            @pl.when(blk < nblk - 1)
            def _():
                for d in idx_cp(blk + 1, 1 - ib):
                    d.wait()

            for b in range(2):
                kk = IDXBLK - 2 + b
                gather(ib, kk, b).wait()
                scat(ib, kk, b)

                @pl.when(blk < nblk - 1)
                def _():
                    gather(1 - ib, b, b).start()

            @pl.when(blk + 2 < nblk)
            def _():
                for d in idx_cp(blk + 2, ib):
                    d.start()

        @pl.loop(0, nblk, step=2)
        def _(blkp):
            for ib in range(2):
                do_block(blkp + ib, ib)

        plsc.subcore_barrier()
        pltpu.sync_copy(acc.at[pl.ds(s * RPS, RPS)],
                        out_hbm.at[c].at[pl.ds(s * RPS, RPS)])

    return sc_agg


_sc_agg_edge = functools.cache(lambda: _make_sc_agg(False))
_sc_agg_feat = functools.cache(lambda: _make_sc_agg(True))


def _bdot(a, b):
    return jnp.dot(a, b, preferred_element_type=jnp.float32)


def _mlp1(x, agg, W1a, b1a, W1b, b1b):
    def body(x_ref, a_ref, wa, ba, wb, bb, out_ref):
        y = x_ref[...] + a_ref[0] + a_ref[1]
        h = jnp.maximum(_bdot(y, wa[...]) + ba[...], 0.0)
        h = jnp.maximum(_bdot(h, wb[...]) + bb[...], 0.0)
        out_ref[0] = h[:, :D]
        out_ref[1] = h[:, D:]

    return pl.pallas_call(
        body,
        grid=(NB,),
        in_specs=[
            pl.BlockSpec((R, D), lambda i: (i, 0)),
            pl.BlockSpec((NC, R, D), lambda i: (0, i, 0)),
            pl.BlockSpec((D, H), lambda i: (0, 0)),
            pl.BlockSpec((1, H), lambda i: (0, 0)),
            pl.BlockSpec((H, H), lambda i: (0, 0)),
            pl.BlockSpec((1, H), lambda i: (0, 0)),
        ],
        out_specs=pl.BlockSpec((NC, R, D), lambda i: (0, i, 0)),
        out_shape=jax.ShapeDtypeStruct((NC, N_NODES, D), jnp.float32),
    )(x, agg, W1a, b1a.reshape(1, H), W1b, b1b.reshape(1, H))


def _mlp2_pool(h1, agg2, batch3, W2a, b2a, W2b, b2b, fcW, fcb):
    def body(h_ref, a_ref, b_ref, wa, ba, wb, bb, wf, bf, out_ref, acc_ref):
        i = pl.program_id(0)
        y = jnp.concatenate([h_ref[0] + a_ref[0], h_ref[1] + a_ref[1]], axis=1)
        t = jnp.maximum(_bdot(y, wa[...]) + ba[...], 0.0)
        t = jnp.maximum(_bdot(t, wb[...]) + bb[...], 0.0)
        seg = b_ref[0, 0, :]
        onehot = (seg[:, None]
                  == lax.broadcasted_iota(jnp.int32, (R, G), 1)
                  ).astype(jnp.float32)
        contrib = lax.dot_general(onehot, t, (((0,), (0,)), ((), ())),
                                  preferred_element_type=jnp.float32)

        @pl.when(i == 0)
        def _():
            acc_ref[...] = jnp.zeros_like(acc_ref)

        acc_ref[...] += contrib

        @pl.when(i == NB - 1)
        def _():
            out_ref[...] = (jnp.dot(acc_ref[...], wf[...],
                                    preferred_element_type=jnp.float32)
                            + bf[...])

    return pl.pallas_call(
        body,
        grid=(NB,),
        in_specs=[
            pl.BlockSpec((NC, R, D), lambda i: (0, i, 0)),
            pl.BlockSpec((NC, R, D), lambda i: (0, i, 0)),
            pl.BlockSpec((1, 1, R), lambda i: (i, 0, 0)),
            pl.BlockSpec((H, H), lambda i: (0, 0)),
            pl.BlockSpec((1, H), lambda i: (0, 0)),
            pl.BlockSpec((H, H), lambda i: (0, 0)),
            pl.BlockSpec((1, H), lambda i: (0, 0)),
            pl.BlockSpec((H, H), lambda i: (0, 0)),
            pl.BlockSpec((1, H), lambda i: (0, 0)),
        ],
        out_specs=pl.BlockSpec((G, H), lambda i: (0, 0)),
        out_shape=jax.ShapeDtypeStruct((G, H), jnp.float32),
        scratch_shapes=[pltpu.VMEM((G, H), jnp.float32)],
    )(h1, agg2, batch3, W2a, b2a.reshape(1, H), W2b, b2b.reshape(1, H),
      fcW, fcb.reshape(1, H))


def kernel(x, edge_index, batch, W1a, b1a, W1b, b1b, W2a, b2a, W2b, b2b,
           fcW, fcb):
    src = edge_index[0]
    dst = edge_index[1]
    pad = EP - N_EDGES
    pad_iota = jnp.arange(pad, dtype=jnp.int32)
    src_p = jnp.concatenate([src, pad_iota % N_NODES]).reshape(NCH, CHUNK)
    dst_p = jnp.concatenate(
        [dst, N_NODES + (pad_iota % 112)]).reshape(NCH, CHUNK)
    zeros = jnp.zeros((N_ACC, D), jnp.float32)

    agg1 = _sc_agg_edge()(x, src_p, dst_p, zeros)
    h1 = _mlp1(x, agg1, W1a, b1a, W1b, b1b)
    agg2 = _sc_agg_feat()(h1, src_p, dst_p, zeros)
    batch3 = batch.reshape(NB, 1, R)
    return _mlp2_pool(h1, agg2, batch3, W2a, b2a, W2b, b2b, fcW, fcb)


# speedup vs baseline: 1.0044x; 1.0044x over previous
import functools

import jax
import jax.numpy as jnp
from jax import lax
from jax.experimental import pallas as pl
from jax.experimental.pallas import tpu as pltpu
from jax.experimental.pallas import tpu_sc as plsc

N_NODES = 10000
N_EDGES = 320000
D = 128
H = 256
G = 128
NC = 2
NS = 16
CHUNK = 128
IDXBLK = 8

EP = 327680
EW1 = EP // (NC * NS)
EW2 = EP // NS
NCH = EP // CHUNK
N_ACC = N_NODES + 112
RPS = N_ACC // NS

R = 2000
NB = N_NODES // R

def _make_sc_agg(feat_split):
    ew = EW2 if feat_split else EW1
    n_chunks = ew // CHUNK
    nblk = n_chunks // IDXBLK
    mesh = plsc.VectorSubcoreMesh(core_axis_name="c", subcore_axis_name="s")

    @functools.partial(
        pl.kernel,
        out_type=jax.ShapeDtypeStruct((NC, N_ACC, D), jnp.float32),
        mesh=mesh,
        scratch_types=[
            pltpu.VMEM((2, IDXBLK, CHUNK), jnp.int32),
            pltpu.VMEM((2, IDXBLK, CHUNK), jnp.int32),
            pltpu.VMEM((CHUNK, D), jnp.float32),
            pltpu.VMEM((CHUNK, D), jnp.float32),
            pltpu.VMEM_SHARED((N_ACC, D), jnp.float32),
            pltpu.SemaphoreType.DMA,
            pltpu.SemaphoreType.DMA,
            pltpu.SemaphoreType.DMA,
            pltpu.SemaphoreType.DMA,
        ],
    )
    def sc_agg(table_hbm, src_hbm, dst_hbm, zeros_hbm, out_hbm,
               idx_s, idx_d, rows0, rows1, acc, sem0, sem1, isem0, isem1):
        c = lax.axis_index("c")
        s = lax.axis_index("s")
        rows = (rows0, rows1)
        sems = (sem0, sem1)
        isems = (isem0, isem1)
        if feat_split:
            tbl = table_hbm.at[c]
            row_base = s * n_chunks
        else:
            tbl = table_hbm
            row_base = (c * NS + s) * n_chunks

        def gather(ib, j, b):
            return pltpu.make_async_copy(tbl.at[idx_s.at[ib].at[j]], rows[b],
                                         sems[b])

        def scat(ib, j, b):
            pltpu.sync_copy(rows[b], acc.at[idx_d.at[ib].at[j]], add=True)

        def idx_cp(blk, ib):
            row0 = row_base + blk * IDXBLK
            return (
                pltpu.make_async_copy(src_hbm.at[pl.ds(row0, IDXBLK)],
                                      idx_s.at[ib], isems[ib]),
                pltpu.make_async_copy(dst_hbm.at[pl.ds(row0, IDXBLK)],
                                      idx_d.at[ib], isems[ib]),
            )

        for d in idx_cp(0, 0):
            d.start()
        if nblk > 1:
            for d in idx_cp(1, 1):
                d.start()
        for d in idx_cp(0, 0):
            d.wait()
        gather(0, 0, 0).start()
        gather(0, 1, 1).start()
        pltpu.sync_copy(zeros_hbm.at[pl.ds(s * RPS, RPS)],
                        acc.at[pl.ds(s * RPS, RPS)])
        plsc.subcore_barrier()

        def do_block(blk, ib):
            @pl.loop(0, IDXBLK - 2, step=2)
            def _(k):
                for b in range(2):
                    kk = k + b
                    gather(ib, kk, b).wait()
                    scat(ib, kk, b)
                    gather(ib, kk + 2, b).start()

            @pl.when(blk < nblk - 1)
            def _():
                for d in idx_cp(blk + 1, 1 - ib):
                    d.wait()

            for b in range(2):
                kk = IDXBLK - 2 + b
                gather(ib, kk, b).wait()
                scat(ib, kk, b)

                @pl.when(blk < nblk - 1)
                def _():
                    gather(1 - ib, b, b).start()

            @pl.when(blk + 2 < nblk)
            def _():
                for d in idx_cp(blk + 2, ib):
                    d.start()

        @pl.loop(0, nblk, step=2)
        def _(blkp):
            for ib in range(2):
                do_block(blkp + ib, ib)

        plsc.subcore_barrier()
        pltpu.sync_copy(acc.at[pl.ds(s * RPS, RPS)],
                        out_hbm.at[c].at[pl.ds(s * RPS, RPS)])

    return sc_agg


_sc_agg_edge = functools.cache(lambda: _make_sc_agg(False))
_sc_agg_feat = functools.cache(lambda: _make_sc_agg(True))


def _bdot(a, b):
    return jnp.dot(a, b, preferred_element_type=jnp.float32)


def _mlp1(x, agg, W1a, b1a, W1b, b1b):
    def body(x_ref, a_ref, wa, ba, wb, bb, out_ref):
        y = x_ref[...] + a_ref[0] + a_ref[1]
        h = jnp.maximum(_bdot(y, wa[...]) + ba[...], 0.0)
        h = jnp.maximum(_bdot(h, wb[...]) + bb[...], 0.0)
        out_ref[0] = h[:, :D]
        out_ref[1] = h[:, D:]

    return pl.pallas_call(
        body,
        grid=(NB,),
        in_specs=[
            pl.BlockSpec((R, D), lambda i: (i, 0)),
            pl.BlockSpec((NC, R, D), lambda i: (0, i, 0)),
            pl.BlockSpec((D, H), lambda i: (0, 0)),
            pl.BlockSpec((1, H), lambda i: (0, 0)),
            pl.BlockSpec((H, H), lambda i: (0, 0)),
            pl.BlockSpec((1, H), lambda i: (0, 0)),
        ],
        out_specs=pl.BlockSpec((NC, R, D), lambda i: (0, i, 0)),
        out_shape=jax.ShapeDtypeStruct((NC, N_NODES, D), jnp.float32),
    )(x, agg, W1a, b1a.reshape(1, H), W1b, b1b.reshape(1, H))


def _mlp2_pool(h1, agg2, batch3, W2a, b2a, W2b, b2b, fcW, fcb):
    def body(h_ref, a_ref, b_ref, wa, ba, wb, bb, wf, bf, out_ref, acc_ref):
        i = pl.program_id(0)
        y = jnp.concatenate([h_ref[0] + a_ref[0], h_ref[1] + a_ref[1]], axis=1)
        t = jnp.maximum(_bdot(y, wa[...]) + ba[...], 0.0)
        t = jnp.maximum(_bdot(t, wb[...]) + bb[...], 0.0)
        seg = b_ref[0, 0, :]
        onehot = (seg[:, None]
                  == lax.broadcasted_iota(jnp.int32, (R, G), 1)
                  ).astype(jnp.float32)
        contrib = lax.dot_general(onehot, t, (((0,), (0,)), ((), ())),
                                  preferred_element_type=jnp.float32)

        @pl.when(i == 0)
        def _():
            acc_ref[...] = jnp.zeros_like(acc_ref)

        acc_ref[...] += contrib

        @pl.when(i == NB - 1)
        def _():
            out_ref[...] = (jnp.dot(acc_ref[...], wf[...],
                                    preferred_element_type=jnp.float32)
                            + bf[...])

    return pl.pallas_call(
        body,
        grid=(NB,),
        in_specs=[
            pl.BlockSpec((NC, R, D), lambda i: (0, i, 0)),
            pl.BlockSpec((NC, R, D), lambda i: (0, i, 0)),
            pl.BlockSpec((1, 1, R), lambda i: (i, 0, 0)),
            pl.BlockSpec((H, H), lambda i: (0, 0)),
            pl.BlockSpec((1, H), lambda i: (0, 0)),
            pl.BlockSpec((H, H), lambda i: (0, 0)),
            pl.BlockSpec((1, H), lambda i: (0, 0)),
            pl.BlockSpec((H, H), lambda i: (0, 0)),
            pl.BlockSpec((1, H), lambda i: (0, 0)),
        ],
        out_specs=pl.BlockSpec((G, H), lambda i: (0, 0)),
        out_shape=jax.ShapeDtypeStruct((G, H), jnp.float32),
        scratch_shapes=[pltpu.VMEM((G, H), jnp.float32)],
    )(h1, agg2, batch3, W2a, b2a.reshape(1, H), W2b, b2b.reshape(1, H),
      fcW, fcb.reshape(1, H))


def kernel(x, edge_index, batch, W1a, b1a, W1b, b1b, W2a, b2a, W2b, b2b,
           fcW, fcb):
    src = edge_index[0]
    dst = edge_index[1]
    pad = EP - N_EDGES
    pad_iota = jnp.arange(pad, dtype=jnp.int32)
    src_p = jnp.concatenate([src, pad_iota % N_NODES]).reshape(NCH, CHUNK)
    dst_p = jnp.concatenate(
        [dst, N_NODES + (pad_iota % 112)]).reshape(NCH, CHUNK)
    zeros = jnp.zeros((N_ACC, D), jnp.float32)

    agg1 = _sc_agg_edge()(x, src_p, dst_p, zeros)
    h1 = _mlp1(x, agg1, W1a, b1a, W1b, b1b)
    agg2 = _sc_agg_feat()(h1, src_p, dst_p, zeros)
    batch3 = batch.reshape(NB, 1, R)
    return _mlp2_pool(h1, agg2, batch3, W2a, b2a, W2b, b2b, fcW, fcb)
